# baseline (speedup 1.0000x reference)
import functools

import jax
import jax.numpy as jnp
from jax import lax
from jax.experimental import pallas as pl
from jax.experimental.pallas import tpu as pltpu
from jax.experimental.pallas import tpu_sc as plsc

_N = 10000
_E = 320000
_NC = 2
_NS = 16
_L = 16
_EB = 4000
_RB = 2000
_CHUNK = {16: 500, 32: 200, 64: 50}
_SUP = 10
_f32 = jnp.float32


def _lr(v):
    return jax.nn.leaky_relu(v, 0.2)


def _fc_body(x_ref, w_ref, b_ref, o_ref):
    o_ref[...] = x_ref[...] @ w_ref[...] + b_ref[...]


def _agg_of(nd0, nd1, w, wp):
    a0 = nd0[:, wp:wp + w] / jnp.maximum(nd0[:, :w], 1e-30)
    a1 = nd1[:, wp:wp + w] / jnp.maximum(nd1[:, :w], 1e-30)
    return jnp.concatenate([a0, a1], axis=1)


def _node_tail(v, w1_ref, b1_ref, wt_ref, bt_ref, f_ref, a0_ref, a1_ref, w, wp):
    f = _lr(v @ w1_ref[...] + b1_ref[...])
    g = f @ wt_ref[...] + bt_ref[...]
    a = jnp.exp(g - jnp.max(g, axis=0, keepdims=True))
    pad = ((0, 0), (0, wp - w))
    f_ref[...] = jnp.pad(f, pad)
    a0_ref[...] = jnp.pad(a[:, :w], pad)
    a1_ref[...] = jnp.pad(a[:, w:], pad)


def _pre_body(x_ref, w1_ref, b1_ref, wt_ref, bt_ref, f_ref, a0_ref, a1_ref,
              *, w, wp):
    _node_tail(x_ref[...], w1_ref, b1_ref, wt_ref, bt_ref,
               f_ref, a0_ref, a1_ref, w, wp)


def _mid_body(nd0_ref, nd1_ref, w1_ref, b1_ref, wt_ref, bt_ref,
              f_ref, a0_ref, a1_ref, *, w, wp):
    agg = _agg_of(nd0_ref[...], nd1_ref[...], w, wp)
    _node_tail(agg, w1_ref, b1_ref, wt_ref, bt_ref, f_ref, a0_ref, a1_ref,
               w, wp)


def _post_body(nd0_ref, nd1_ref, wm_ref, bm_ref, wlo_ref,
               blo_ref, x_ref, wsc_ref, bsc_ref, o_ref, *, w, wp):
    agg = _agg_of(nd0_ref[...], nd1_ref[...], w, wp)
    f3 = agg @ wm_ref[...] + bm_ref[...]
    o_ref[...] = _lr(f3 @ wlo_ref[...] + blo_ref[...]
                     + x_ref[...] @ wsc_ref[...] + bsc_ref[...])


def _edge1_body(pd_ref, ps_ref, wpe_ref, bpe_ref, wbot_ref,
                pe_ref, h_ref, hm_ref, hs_ref, *, nsteps):
    pd = pd_ref[...][:, 0:3]
    ps = ps_ref[...][:, 0:3]
    d = pd - ps
    dist = jnp.sqrt(jnp.sum(d * d, axis=1, keepdims=True) + 1e-12)
    eb = pd.shape[0]
    rel = jnp.concatenate([pd, ps, d, dist, jnp.zeros((eb, 6), _f32)], axis=1)
    pe = _lr(rel @ wpe_ref[...] + bpe_ref[...])
    h = pe @ wbot_ref[...]
    pe_ref[...] = pe
    h_ref[...] = h
    bm = jnp.max(h, axis=0, keepdims=True)
    i = pl.program_id(0)

    @pl.when(i == 0)
    def _():
        hs_ref[...] = jnp.broadcast_to(bm, hs_ref.shape)

    @pl.when(i > 0)
    def _():
        hs_ref[...] = jnp.maximum(hs_ref[...], bm)

    @pl.when(i == nsteps - 1)
    def _():
        hm_ref[...] = hs_ref[...]


def _edge2_body(h_ref, hm_ref, e0_ref, e1_ref, *, w, wp):
    mh = jnp.max(hm_ref[...], axis=0, keepdims=True)
    eh = jnp.exp(h_ref[...] - mh)
    pad = ((0, 0), (0, wp - w))
    e0_ref[...] = jnp.pad(eh[:, :w], pad)
    e1_ref[...] = jnp.pad(eh[:, w:], pad)


def _dec_body(x1_ref, x2_ref, x3_ref, x4_ref, x5_ref,
              w11, b11, w12, b12, w2, b2, w31, b31, w32, b32,
              w41, b41, w42, b42, w51, b51, w52, b52,
              wf2, bf2, wf3, bf3, wf4, bf4, o_ref):
    relu = jax.nn.relu
    x5 = x5_ref[...]
    x6 = relu(x5 @ w11[...] + b11[...]) @ w12[...] + b12[...]
    x7 = relu(x6 @ w2[...] + b2[...])
    x7c = jnp.concatenate([x4_ref[...], x7], axis=1)
    x8 = relu(x7c @ w31[...] + b31[...]) @ w32[...] + b32[...]
    x8c = jnp.concatenate([x3_ref[...], x8], axis=1)
    x9 = relu(x8c @ w41[...] + b41[...]) @ w42[...] + b42[...]
    x9c = jnp.concatenate([x2_ref[...], x9], axis=1)
    x10 = relu(x9c @ w51[...] + b51[...]) @ w52[...] + b52[...]
    x10c = jnp.concatenate([x1_ref[...], x10], axis=1)
    x11 = x10c @ wf2[...] + bf2[...]
    x12 = x11 @ wf3[...] + bf3[...]
    o_ref[...] = jax.nn.sigmoid(x12 @ wf4[...] + bf4[...])


_MESH = dict(core_axis_name="c", subcore_axis_name="s")
_SC_PARAMS = pltpu.CompilerParams(use_tc_tiling_on_sc=False)


@functools.lru_cache(maxsize=None)
def _sc_posgather():
    cg = 1000
    ew = _E // (_NC * _NS)
    mesh = plsc.VectorSubcoreMesh(**_MESH)

    @functools.partial(
        pl.kernel,
        out_type=[jax.ShapeDtypeStruct((_E, 16), _f32)] * 2,
        mesh=mesh,
        compiler_params=_SC_PARAMS,
        scratch_types=[
            pltpu.VMEM((cg,), jnp.int32),
            pltpu.VMEM((cg, 16), _f32),
            pltpu.SemaphoreType.DMA,
        ],
    )
    def k(pos16_h, src_h, dst_h, posd_h, poss_h, idx_v, rows_v, sem):
        cid = lax.axis_index("c")
        sid = lax.axis_index("s")
        base = (sid * _NC + cid) * ew

        def chunk(kk, carry):
            b = base + kk * cg
            pltpu.sync_copy(dst_h.at[pl.ds(b, cg)], idx_v)
            pltpu.async_copy(pos16_h.at[idx_v], rows_v, sem).wait()
            pltpu.sync_copy(rows_v, posd_h.at[pl.ds(b, cg)])
            pltpu.sync_copy(src_h.at[pl.ds(b, cg)], idx_v)
            pltpu.async_copy(pos16_h.at[idx_v], rows_v, sem).wait()
            pltpu.sync_copy(rows_v, poss_h.at[pl.ds(b, cg)])
            return carry

        lax.fori_loop(0, ew // cg, chunk, 0)

    return k


@functools.lru_cache(maxsize=None)
def _sc_pool(wp):
    cc = _CHUNK[wp]
    wp2 = 2 * wp
    et = _E // _NS
    nck = et // cc
    nsup = nck // _SUP
    rz = _N // _NS
    nj = wp // _L
    mesh = plsc.VectorSubcoreMesh(**_MESH)

    @functools.partial(
        pl.kernel,
        out_type=[jax.ShapeDtypeStruct((_N, wp2), _f32)] * 2,
        mesh=mesh,
        compiler_params=_SC_PARAMS,
        scratch_types=[
            pltpu.VMEM((2, _SUP, cc), jnp.int32),
            pltpu.VMEM((2, _SUP, cc), jnp.int32),
            pltpu.VMEM((cc, wp), _f32),
            pltpu.VMEM((cc, wp), _f32),
            pltpu.VMEM((cc, wp), _f32),
            pltpu.VMEM((cc, wp2), _f32),
            pltpu.VMEM((cc, wp), _f32),
            pltpu.VMEM((cc, wp), _f32),
            pltpu.VMEM((cc, wp), _f32),
            pltpu.VMEM((cc, wp2), _f32),
            pltpu.SemaphoreType.DMA,
            pltpu.SemaphoreType.DMA,
            pltpu.SemaphoreType.DMA,
            pltpu.SemaphoreType.DMA,
            pltpu.SemaphoreType.DMA,
            pltpu.SemaphoreType.DMA,
            pltpu.VMEM_SHARED((_N, wp2), _f32),
        ],
    )
    def k(src2_h, dst2_h, a0_h, a1_h, f_h, pe_h, eh0_h, eh1_h, zer_h,
          nd0_h, nd1_h,
          idxs, idxd, ba0, b30, be0, so0, ba1, b31, be1, so1,
          semi0, semi1, seml0, seml1, sems0, sems1, sp):
        cid = lax.axis_index("c")
        sid = lax.axis_index("s")
        r0 = sid * rz
        pltpu.sync_copy(zer_h, sp.at[pl.ds(r0, rz)])
        plsc.subcore_barrier()

        base0 = sid * et
        crow0 = sid * nck
        semis = (semi0, semi1)

        def issue_super(sb, sslot):
            r = crow0 + sb * _SUP
            pltpu.async_copy(src2_h.at[pl.ds(r, _SUP)], idxs.at[sslot],
                             semis[sslot])
            pltpu.async_copy(dst2_h.at[pl.ds(r, _SUP)], idxd.at[sslot],
                             semis[sslot])

        def drain_super(sslot):
            for _ in range(2):
                pltpu.make_async_copy(src2_h.at[pl.ds(0, _SUP)],
                                      idxs.at[0], semis[sslot]).wait()

        def issue_data(sb, sslot, cj, ba, b3, be, sem):
            b = base0 + (sb * _SUP + cj) * cc
            irow = idxs.at[sslot, cj]

            @pl.when(cid == 0)
            def _():
                pltpu.async_copy(a0_h.at[irow], ba, sem)
                pltpu.async_copy(f_h.at[irow], b3, sem)
                pltpu.async_copy(eh0_h.at[pl.ds(b, cc)], be, sem)

            @pl.when(cid == 1)
            def _():
                pltpu.async_copy(a1_h.at[irow], ba, sem)
                pltpu.async_copy(pe_h.at[pl.ds(b, cc)], b3, sem)
                pltpu.async_copy(eh1_h.at[pl.ds(b, cc)], be, sem)

        def drain(sem, n):
            for _ in range(n):
                pltpu.make_async_copy(eh0_h.at[pl.ds(0, cc)], ba0, sem).wait()

        def drain_scat(sem):
            pltpu.make_async_copy(so0, sp.at[pl.ds(0, cc)], sem).wait()

        def compute(ba, b3, be, so):
            def row(i, c2):
                for j in range(nj):
                    q = j * _L
                    e = ba[i, pl.ds(q, _L)] * be[i, pl.ds(q, _L)]
                    so[i, pl.ds(q, _L)] = e
                    so[i, pl.ds(wp + q, _L)] = e * b3[i, pl.ds(q, _L)]
                return c2

            lax.fori_loop(0, cc, row, 0, unroll=2)

        def issue_scatter(sslot, cj, so, sem):
            pltpu.async_copy(so, sp.at[idxd.at[sslot, cj]], sem, add=True)

        def process_super(sb, sslot):
            @pl.when(sb > 0)
            def _():
                drain_scat(sems1)

            @pl.when(sb + 1 < nsup)
            def _():
                issue_super(sb + 1, 1 - sslot)

            drain_super(sslot)
            issue_data(sb, sslot, 0, ba0, b30, be0, seml0)
            for j in range(_SUP // 2):
                ca, cb = 2 * j, 2 * j + 1
                if j > 0:
                    drain_scat(sems1)
                issue_data(sb, sslot, cb, ba1, b31, be1, seml1)
                drain(seml0, 3)
                compute(ba0, b30, be0, so0)
                issue_scatter(sslot, ca, so0, sems0)
                drain(seml1, 3)
                compute(ba1, b31, be1, so1)
                issue_scatter(sslot, cb, so1, sems1)
                drain_scat(sems0)
                if j < _SUP // 2 - 1:
                    issue_data(sb, sslot, ca + 2, ba0, b30, be0, seml0)

        issue_super(0, 0)

        def outer(p, carry):
            process_super(2 * p, 0)
            process_super(2 * p + 1, 1)
            return carry

        lax.fori_loop(0, nsup // 2, outer, 0)
        drain_scat(sems1)
        plsc.subcore_barrier()

        @pl.when(cid == 0)
        def _():
            pltpu.sync_copy(sp.at[pl.ds(r0, rz)], nd0_h.at[pl.ds(r0, rz)])

        @pl.when(cid == 1)
        def _():
            pltpu.sync_copy(sp.at[pl.ds(r0, rz)], nd1_h.at[pl.ds(r0, rz)])

    return k


def _padw(wb, rows, cols):
    return jnp.zeros((rows, cols), _f32).at[:wb.shape[0], :wb.shape[1]].set(wb)


def _edge_stage(posd, poss, wpe, bpe, wbot, wp, c):
    nsteps = _E // _EB
    cmap = lambda i: (0, 0)
    rmap = lambda i: (i, 0)
    pe, h, hm = pl.pallas_call(
        functools.partial(_edge1_body, nsteps=nsteps),
        grid=(nsteps,),
        in_specs=[
            pl.BlockSpec((_EB, 16), rmap),
            pl.BlockSpec((_EB, 16), rmap),
            pl.BlockSpec((16, wp), cmap),
            pl.BlockSpec((1, wp), cmap),
            pl.BlockSpec((wp, c), cmap),
        ],
        out_specs=[
            pl.BlockSpec((_EB, wp), rmap),
            pl.BlockSpec((_EB, c), rmap),
            pl.BlockSpec((8, c), cmap),
        ],
        out_shape=[
            jax.ShapeDtypeStruct((_E, wp), _f32),
            jax.ShapeDtypeStruct((_E, c), _f32),
            jax.ShapeDtypeStruct((8, c), _f32),
        ],
        scratch_shapes=[pltpu.VMEM((8, c), _f32)],
    )(posd, poss, wpe, bpe, wbot)
    eh0, eh1 = pl.pallas_call(
        functools.partial(_edge2_body, w=c // 2, wp=wp),
        grid=(nsteps,),
        in_specs=[
            pl.BlockSpec((_EB, c), rmap),
            pl.BlockSpec((8, c), cmap),
        ],
        out_specs=[
            pl.BlockSpec((_EB, wp), rmap),
            pl.BlockSpec((_EB, wp), rmap),
        ],
        out_shape=[
            jax.ShapeDtypeStruct((_E, wp), _f32),
            jax.ShapeDtypeStruct((_E, wp), _f32),
        ],
    )(h, hm)
    return pe, eh0, eh1


def _resblock(x_in, posd, poss, e2, zer, p, c):
    w = c // 2
    wp = max(_L, w)
    src2, dst2 = e2[_CHUNK[wp]]
    win, bin_ = p["lin_in"]
    wa1, ba1 = p["att1"]
    wa2, ba2 = p["att2"]
    wp1, bp1 = p["pos1"]
    wp2, bp2 = p["pos2"]
    wm1, bm1 = p["mlp1"]
    wm2, bm2 = p["mlp2"]
    wlo, blo = p["lin_out"]
    wsc, bsc = p["shortcut"]

    node_out = [jax.ShapeDtypeStruct((_N, wp), _f32)] * 3
    f_pad, a0, a1 = pl.pallas_call(
        functools.partial(_pre_body, w=w, wp=wp),
        out_shape=node_out,
    )(x_in, win, bin_[None, :], wa1[:w], ba1[None, :])

    pe1, eh0, eh1 = _edge_stage(posd, poss, _padw(wp1, 16, wp),
                                _padw(bp1[None, :], 1, wp),
                                _padw(wa1[w:], wp, c), wp, c)
    nd0, nd1 = _sc_pool(wp)(src2, dst2, a0, a1, f_pad, pe1, eh0, eh1, zer)

    f2_pad, a0b, a1b = pl.pallas_call(
        functools.partial(_mid_body, w=w, wp=wp),
        out_shape=node_out,
    )(nd0, nd1, wm1, bm1[None, :], wa2[:w], ba2[None, :])

    pe2, eh0b, eh1b = _edge_stage(posd, poss, _padw(wp2, 16, wp),
                                  _padw(bp2[None, :], 1, wp),
                                  _padw(wa2[w:], wp, c), wp, c)
    nd0b, nd1b = _sc_pool(wp)(src2, dst2, a0b, a1b, f2_pad, pe2,
                              eh0b, eh1b, zer)

    x_out = pl.pallas_call(
        functools.partial(_post_body, w=w, wp=wp),
        out_shape=jax.ShapeDtypeStruct((_N, 2 * c), _f32),
    )(nd0b, nd1b, wm2, bm2[None, :], wlo, blo[None, :],
      x_in, wsc, bsc[None, :])
    return x_out


def kernel(x, pos, ei, params):
    src = ei[0]
    dst = ei[1]
    e2 = {cc: (src.reshape(_E // cc, cc), dst.reshape(_E // cc, cc))
          for cc in set(_CHUNK.values())}
    pos16 = jnp.zeros((_N, 16), _f32).at[:, :3].set(pos)
    posd, poss = _sc_posgather()(pos16, src, dst)

    wfc, bfc = params["fc"]
    x1 = pl.pallas_call(
        _fc_body,
        out_shape=jax.ShapeDtypeStruct((_N, 16), _f32),
    )(x, wfc, bfc[None, :])

    zers = {wpv: jnp.zeros((_N // _NS, 2 * wpv), _f32) for wpv in (16, 32, 64)}
    x2 = _resblock(x1, posd, poss, e2, zers[16], params["rb1"], 16)
    x3 = _resblock(x2, posd, poss, e2, zers[16], params["rb2"], 32)
    x4 = _resblock(x3, posd, poss, e2, zers[32], params["rb3"], 64)
    x5 = _resblock(x4, posd, poss, e2, zers[64], params["rb4"], 128)

    dw = []
    for name in ("mlp1_1", "mlp1_2", "mlp2", "mlp3_1", "mlp3_2",
                 "mlp4_1", "mlp4_2", "mlp5_1", "mlp5_2",
                 "fc2", "fc3", "fc4"):
        wv, bv = params[name]
        dw += [wv, bv[None, :]]

    rmap = lambda i: (i, 0)
    cmap = lambda i: (0, 0)
    nsteps = _N // _RB
    xspecs = [pl.BlockSpec((_RB, s), rmap) for s in (16, 32, 64, 128, 256)]
    wspecs = [pl.BlockSpec(wv.shape, cmap) for wv in dw]
    out = pl.pallas_call(
        _dec_body,
        grid=(nsteps,),
        in_specs=xspecs + wspecs,
        out_specs=pl.BlockSpec((_RB, 13), rmap),
        out_shape=jax.ShapeDtypeStruct((_N, 13), _f32),
    )(x1, x2, x3, x4, x5, *dw)
    return out
